# baseline (speedup 1.0000x reference)
import functools

import jax
import jax.numpy as jnp
from jax import lax
from jax.experimental import pallas as pl
from jax.experimental.pallas import tpu as pltpu
from jax.experimental.pallas import tpu_sc as plsc

M = 1000000
B = 16384
D = 32
H1 = 64
H2 = 32

RB = 2048
G = 124
S = RB * G

NC = 2
NS = 16
NW = NC * NS
BPW = B // NW
CH = 128
NCH = BPW // CH

BLK = 2048


def _relayout_body(in0, in1, in2, in3, out_ref):
  cat = jnp.concatenate(
      [in0[...], in1[...], in2[...], in3[...]], axis=0)
  u = lax.bitcast_convert_type(cat.T, jnp.uint32)
  b = (u + jnp.uint32(0x7FFF) + ((u >> jnp.uint32(16)) & jnp.uint32(1))
       ) >> jnp.uint32(16)
  packed = b[:, :64] | (b[:, 64:] << jnp.uint32(16))
  out_ref[...] = lax.bitcast_convert_type(packed, jnp.float32)


def _tc_relayout(tt):
  return pl.pallas_call(
      _relayout_body,
      grid=(G,),
      in_specs=[pl.BlockSpec(
          (32, RB),
          lambda g, k=k: (0, jnp.minimum((k * S) // RB + g, M // RB)))
                for k in range(4)],
      out_specs=pl.BlockSpec((RB, 64), lambda g: (g, 0)),
      out_shape=jax.ShapeDtypeStruct((S, 64), jnp.float32),
  )(tt, tt, tt, tt)


def _sc_gather(p2d, lin):
  mesh = plsc.VectorSubcoreMesh(core_axis_name="c", subcore_axis_name="s")

  @functools.partial(
      pl.kernel,
      out_type=jax.ShapeDtypeStruct((B, 64), jnp.float32),
      mesh=mesh,
      compiler_params=pltpu.CompilerParams(use_tc_tiling_on_sc=False),
      scratch_types=[
          pltpu.VMEM((NCH, CH), jnp.int32),
          pltpu.VMEM((BPW, 64), jnp.float32),
          pltpu.SemaphoreType.DMA,
      ],
  )
  def gather_kernel(p_hbm, lin_hbm, x_hbm, idx_v, rows_v, sem):
    wid = lax.axis_index("s") * NC + lax.axis_index("c")
    base = wid * BPW
    pltpu.sync_copy(p_hbm.at[pl.ds(wid * NCH, NCH)], idx_v)
    cps = [
        pltpu.async_copy(lin_hbm.at[idx_v.at[j]],
                         rows_v.at[pl.ds(j * CH, CH)], sem)
        for j in range(NCH)
    ]
    for c in cps:
      c.wait()
    pltpu.sync_copy(rows_v, x_hbm.at[pl.ds(base, BPW)])

  return gather_kernel(p2d, lin)


def _mlp_body(xu_ref, xb_ref, offu_ref, offb_ref, w1u_ref, w1b_ref, b1_ref,
              w2t_ref, b2_ref, o_ref):
  lane = lax.broadcasted_iota(jnp.int32, (BLK, 128), 1)
  offu = offu_ref[...]
  offb = offb_ref[...]

  def unpack(packed_f32):
    u = lax.bitcast_convert_type(packed_f32, jnp.uint32)
    lo = u << jnp.uint32(16)
    hi = u & jnp.uint32(0xFFFF0000)
    pair = jnp.concatenate([lo, hi], axis=1)
    return lax.bitcast_convert_type(pair, jnp.float32)

  xu = jnp.where((lane >= offu) & (lane < offu + D), unpack(xu_ref[...]), 0.0)
  xb = jnp.where((lane >= offb) & (lane < offb + D), unpack(xb_ref[...]), 0.0)
  h = lax.dot_general(xu, w1u_ref[...], (((1,), (0,)), ((), ())),
                      preferred_element_type=jnp.float32)
  h = h + lax.dot_general(xb, w1b_ref[...], (((1,), (0,)), ((), ())),
                          preferred_element_type=jnp.float32)
  h = jnp.maximum(h + b1_ref[...], 0.0)
  o = lax.dot_general(h, w2t_ref[...], (((1,), (0,)), ((), ())),
                      preferred_element_type=jnp.float32)
  o_ref[...] = jnp.maximum(o + b2_ref[...], 0.0)


def _tc_mlp(xu, xb, offu, offb, w1u4t, w1b4t, b1r, w2t, b2r):
  grid = (B // BLK,)
  return pl.pallas_call(
      _mlp_body,
      grid=grid,
      in_specs=[
          pl.BlockSpec((BLK, 64), lambda i: (i, 0)),
          pl.BlockSpec((BLK, 64), lambda i: (i, 0)),
          pl.BlockSpec((BLK, 1), lambda i: (i, 0)),
          pl.BlockSpec((BLK, 1), lambda i: (i, 0)),
          pl.BlockSpec((128, H1), lambda i: (0, 0)),
          pl.BlockSpec((128, H1), lambda i: (0, 0)),
          pl.BlockSpec((1, H1), lambda i: (0, 0)),
          pl.BlockSpec((H1, H2), lambda i: (0, 0)),
          pl.BlockSpec((1, H2), lambda i: (0, 0)),
      ],
      out_specs=pl.BlockSpec((BLK, H2), lambda i: (i, 0)),
      out_shape=jax.ShapeDtypeStruct((B, H2), jnp.float32),
  )(xu, xb, offu, offb, w1u4t, w1b4t, b1r, w2t, b2r)


def _split(idx):
  idx = idx.astype(jnp.int32)
  k = ((idx >= S).astype(jnp.int32) + (idx >= 2 * S).astype(jnp.int32)
       + (idx >= 3 * S).astype(jnp.int32))
  p = idx - k * S
  return p.reshape(NW * NCH, CH), (k * D).reshape(B, 1)


def kernel(user_id, book_id, user_table, book_table, W1, b1, W2, b2):
  pu2d, offu = _split(user_id)
  pb2d, offb = _split(book_id)
  lin_u = _tc_relayout(user_table.T)
  xu = _sc_gather(pu2d, lin_u)
  lin_b = _tc_relayout(book_table.T)
  xb = _sc_gather(pb2d, lin_b)
  w1u4t = jnp.tile(W1[:, :D], (1, 4)).T
  w1b4t = jnp.tile(W1[:, D:], (1, 4)).T
  return _tc_mlp(xu, xb, offu, offb, w1u4t, w1b4t, b1.reshape(1, H1),
                 W2.T, b2.reshape(1, H2))


# speedup vs baseline: 2.3326x; 2.3326x over previous
import functools

import jax
import jax.numpy as jnp
from jax import lax
from jax.experimental import pallas as pl
from jax.experimental.pallas import tpu as pltpu
from jax.experimental.pallas import tpu_sc as plsc

M = 1000000
B = 16384
D = 32
H1 = 64
H2 = 32

RB = 2048
G = 124
S = RB * G

NC = 2
NS = 16
NW = NC * NS
BPW = B // NW
CH = 128
NCH = BPW // CH

BLK = 2048


def _round_bf16(u):
  return (u + jnp.uint32(0x7FFF) + ((u >> jnp.uint32(16)) & jnp.uint32(1))
          ) >> jnp.uint32(16)


def _relayout_body(u0, u1, u2, u3, b0, b1, b2, b3, out_ref):
  cat_u = jnp.concatenate(
      [u0[...], u1[...], u2[...], u3[...]], axis=0)
  cat_b = jnp.concatenate(
      [b0[...], b1[...], b2[...], b3[...]], axis=0)
  bu = _round_bf16(lax.bitcast_convert_type(cat_u.T, jnp.uint32))
  bb = _round_bf16(lax.bitcast_convert_type(cat_b.T, jnp.uint32))
  packed = bu | (bb << jnp.uint32(16))
  out_ref[...] = lax.bitcast_convert_type(packed, jnp.float32)


def _tc_relayout(ttu, ttb):
  specs = [pl.BlockSpec(
      (32, RB),
      lambda g, k=k: (0, jnp.minimum((k * S) // RB + g, M // RB)))
      for k in range(4)]
  return pl.pallas_call(
      _relayout_body,
      grid=(G,),
      in_specs=specs + specs,
      out_specs=pl.BlockSpec((RB, 128), lambda g: (g, 0)),
      out_shape=jax.ShapeDtypeStruct((S, 128), jnp.float32),
  )(ttu, ttu, ttu, ttu, ttb, ttb, ttb, ttb)


def _sc_gather(p2d, lin):
  mesh = plsc.VectorSubcoreMesh(core_axis_name="c", subcore_axis_name="s")

  @functools.partial(
      pl.kernel,
      out_type=jax.ShapeDtypeStruct((B, 128), jnp.float32),
      mesh=mesh,
      compiler_params=pltpu.CompilerParams(use_tc_tiling_on_sc=False),
      scratch_types=[
          pltpu.VMEM((NCH, CH), jnp.int32),
          pltpu.VMEM((BPW, 128), jnp.float32),
          pltpu.SemaphoreType.DMA,
      ],
  )
  def gather_kernel(p_hbm, lin_hbm, x_hbm, idx_v, rows_v, sem):
    wid = lax.axis_index("s") * NC + lax.axis_index("c")
    base = wid * BPW
    pltpu.sync_copy(p_hbm.at[pl.ds(wid * NCH, NCH)], idx_v)
    cps = [
        pltpu.async_copy(lin_hbm.at[idx_v.at[j]],
                         rows_v.at[pl.ds(j * CH, CH)], sem)
        for j in range(NCH)
    ]
    for c in cps:
      c.wait()
    pltpu.sync_copy(rows_v, x_hbm.at[pl.ds(base, BPW)])

  return gather_kernel(p2d, lin)


def _mlp_body(xu_ref, xb_ref, offu_ref, offb_ref, w1u_ref, w1b_ref, b1_ref,
              w2t_ref, b2_ref, o_ref):
  lane = lax.broadcasted_iota(jnp.int32, (BLK, 128), 1)
  offu = offu_ref[...]
  offb = offb_ref[...]
  uu = lax.bitcast_convert_type(xu_ref[...], jnp.uint32)
  ub = lax.bitcast_convert_type(xb_ref[...], jnp.uint32)
  xu_vals = lax.bitcast_convert_type(uu << jnp.uint32(16), jnp.float32)
  xb_vals = lax.bitcast_convert_type(ub & jnp.uint32(0xFFFF0000), jnp.float32)
  xu = jnp.where((lane >= offu) & (lane < offu + D), xu_vals, 0.0)
  xb = jnp.where((lane >= offb) & (lane < offb + D), xb_vals, 0.0)
  h = lax.dot_general(xu, w1u_ref[...], (((1,), (0,)), ((), ())),
                      preferred_element_type=jnp.float32)
  h = h + lax.dot_general(xb, w1b_ref[...], (((1,), (0,)), ((), ())),
                          preferred_element_type=jnp.float32)
  h = jnp.maximum(h + b1_ref[...], 0.0)
  o = lax.dot_general(h, w2t_ref[...], (((1,), (0,)), ((), ())),
                      preferred_element_type=jnp.float32)
  o_ref[...] = jnp.maximum(o + b2_ref[...], 0.0)


def _tc_mlp(xu, xb, offu, offb, w1u4t, w1b4t, b1r, w2t, b2r):
  grid = (B // BLK,)
  return pl.pallas_call(
      _mlp_body,
      grid=grid,
      in_specs=[
          pl.BlockSpec((BLK, 128), lambda i: (i, 0)),
          pl.BlockSpec((BLK, 128), lambda i: (i, 0)),
          pl.BlockSpec((BLK, 1), lambda i: (i, 0)),
          pl.BlockSpec((BLK, 1), lambda i: (i, 0)),
          pl.BlockSpec((128, H1), lambda i: (0, 0)),
          pl.BlockSpec((128, H1), lambda i: (0, 0)),
          pl.BlockSpec((1, H1), lambda i: (0, 0)),
          pl.BlockSpec((H1, H2), lambda i: (0, 0)),
          pl.BlockSpec((1, H2), lambda i: (0, 0)),
      ],
      out_specs=pl.BlockSpec((BLK, H2), lambda i: (i, 0)),
      out_shape=jax.ShapeDtypeStruct((B, H2), jnp.float32),
  )(xu, xb, offu, offb, w1u4t, w1b4t, b1r, w2t, b2r)


def _split(idx):
  idx = idx.astype(jnp.int32)
  k = ((idx >= S).astype(jnp.int32) + (idx >= 2 * S).astype(jnp.int32)
       + (idx >= 3 * S).astype(jnp.int32))
  p = idx - k * S
  return p.reshape(NW * NCH, CH), (k * D).reshape(B, 1)


def kernel(user_id, book_id, user_table, book_table, W1, b1, W2, b2):
  pu2d, offu = _split(user_id)
  pb2d, offb = _split(book_id)
  lin = _tc_relayout(user_table.T, book_table.T)
  xu = _sc_gather(pu2d, lin)
  xb = _sc_gather(pb2d, lin)
  w1u4t = jnp.tile(W1[:, :D], (1, 4)).T
  w1b4t = jnp.tile(W1[:, D:], (1, 4)).T
  return _tc_mlp(xu, xb, offu, offb, w1u4t, w1b4t, b1.reshape(1, H1),
                 W2.T, b2.reshape(1, H2))


# speedup vs baseline: 2.7381x; 1.1738x over previous
import functools

import jax
import jax.numpy as jnp
from jax import lax
from jax.experimental import pallas as pl
from jax.experimental.pallas import tpu as pltpu
from jax.experimental.pallas import tpu_sc as plsc

M = 1000000
B = 16384
D = 32
H1 = 64
H2 = 32

RB = 4096
G = 62
S = RB * G

NC = 2
NS = 16
NW = NC * NS
BPW = B // NW
CH = 128
NCH = BPW // CH

BLK = 2048


def _round_bf16(u):
  return (u + jnp.uint32(0x7FFF) + ((u >> jnp.uint32(16)) & jnp.uint32(1))
          ) >> jnp.uint32(16)


def _relayout_body(u0, u1, u2, u3, b0, b1, b2, b3, out_ref):
  cat_u = jnp.concatenate(
      [u0[...], u1[...], u2[...], u3[...]], axis=0)
  cat_b = jnp.concatenate(
      [b0[...], b1[...], b2[...], b3[...]], axis=0)
  bu = _round_bf16(lax.bitcast_convert_type(cat_u.T, jnp.uint32))
  bb = _round_bf16(lax.bitcast_convert_type(cat_b.T, jnp.uint32))
  packed = bu | (bb << jnp.uint32(16))
  out_ref[...] = lax.bitcast_convert_type(packed, jnp.float32)


def _tc_relayout(ttu, ttb):
  specs = [pl.BlockSpec(
      (32, RB),
      lambda g, k=k: (0, jnp.minimum((k * S) // RB + g, M // RB)))
      for k in range(4)]
  return pl.pallas_call(
      _relayout_body,
      grid=(G,),
      in_specs=specs + specs,
      out_specs=pl.BlockSpec((RB, 128), lambda g: (g, 0)),
      out_shape=jax.ShapeDtypeStruct((S, 128), jnp.float32),
  )(ttu, ttu, ttu, ttu, ttb, ttb, ttb, ttb)


def _sc_gather(p2d, lin):
  mesh = plsc.VectorSubcoreMesh(core_axis_name="c", subcore_axis_name="s")

  @functools.partial(
      pl.kernel,
      out_type=jax.ShapeDtypeStruct((B, 128), jnp.float32),
      mesh=mesh,
      compiler_params=pltpu.CompilerParams(use_tc_tiling_on_sc=False),
      scratch_types=[
          pltpu.VMEM((NCH, CH), jnp.int32),
          pltpu.VMEM((BPW, 128), jnp.float32),
          pltpu.SemaphoreType.DMA,
      ],
  )
  def gather_kernel(p_hbm, lin_hbm, x_hbm, idx_v, rows_v, sem):
    wid = lax.axis_index("s") * NC + lax.axis_index("c")
    base = wid * BPW
    pltpu.sync_copy(p_hbm.at[pl.ds(wid * NCH, NCH)], idx_v)
    cps = [
        pltpu.async_copy(lin_hbm.at[idx_v.at[j]],
                         rows_v.at[pl.ds(j * CH, CH)], sem)
        for j in range(NCH)
    ]
    for c in cps:
      c.wait()
    pltpu.sync_copy(rows_v, x_hbm.at[pl.ds(base, BPW)])

  return gather_kernel(p2d, lin)


def _mlp_body(xu_ref, xb_ref, offu_ref, offb_ref, w1u_ref, w1b_ref, b1_ref,
              w2t_ref, b2_ref, o_ref):
  lane = lax.broadcasted_iota(jnp.int32, (BLK, 128), 1)
  offu = offu_ref[...]
  offb = offb_ref[...]
  uu = lax.bitcast_convert_type(xu_ref[...], jnp.uint32)
  ub = lax.bitcast_convert_type(xb_ref[...], jnp.uint32)
  xu_vals = lax.bitcast_convert_type(uu << jnp.uint32(16), jnp.float32)
  xb_vals = lax.bitcast_convert_type(ub & jnp.uint32(0xFFFF0000), jnp.float32)
  xu = jnp.where((lane >= offu) & (lane < offu + D), xu_vals, 0.0)
  xb = jnp.where((lane >= offb) & (lane < offb + D), xb_vals, 0.0)
  h = lax.dot_general(xu, w1u_ref[...], (((1,), (0,)), ((), ())),
                      preferred_element_type=jnp.float32)
  h = h + lax.dot_general(xb, w1b_ref[...], (((1,), (0,)), ((), ())),
                          preferred_element_type=jnp.float32)
  h = jnp.maximum(h + b1_ref[...], 0.0)
  o = lax.dot_general(h, w2t_ref[...], (((1,), (0,)), ((), ())),
                      preferred_element_type=jnp.float32)
  o_ref[...] = jnp.maximum(o + b2_ref[...], 0.0)


def _tc_mlp(xu, xb, offu, offb, w1u4t, w1b4t, b1r, w2t, b2r):
  grid = (B // BLK,)
  return pl.pallas_call(
      _mlp_body,
      grid=grid,
      in_specs=[
          pl.BlockSpec((BLK, 128), lambda i: (i, 0)),
          pl.BlockSpec((BLK, 128), lambda i: (i, 0)),
          pl.BlockSpec((BLK, 1), lambda i: (i, 0)),
          pl.BlockSpec((BLK, 1), lambda i: (i, 0)),
          pl.BlockSpec((128, H1), lambda i: (0, 0)),
          pl.BlockSpec((128, H1), lambda i: (0, 0)),
          pl.BlockSpec((1, H1), lambda i: (0, 0)),
          pl.BlockSpec((H1, H2), lambda i: (0, 0)),
          pl.BlockSpec((1, H2), lambda i: (0, 0)),
      ],
      out_specs=pl.BlockSpec((BLK, H2), lambda i: (i, 0)),
      out_shape=jax.ShapeDtypeStruct((B, H2), jnp.float32),
  )(xu, xb, offu, offb, w1u4t, w1b4t, b1r, w2t, b2r)


def _split(idx):
  idx = idx.astype(jnp.int32)
  k = ((idx >= S).astype(jnp.int32) + (idx >= 2 * S).astype(jnp.int32)
       + (idx >= 3 * S).astype(jnp.int32))
  p = idx - k * S
  return p.reshape(NW * NCH, CH), (k * D).reshape(B, 1)


def kernel(user_id, book_id, user_table, book_table, W1, b1, W2, b2):
  pu2d, offu = _split(user_id)
  pb2d, offb = _split(book_id)
  lin = _tc_relayout(user_table.T, book_table.T)
  xu = _sc_gather(pu2d, lin)
  xb = _sc_gather(pb2d, lin)
  w1u4t = jnp.tile(W1[:, :D], (1, 4)).T
  w1b4t = jnp.tile(W1[:, D:], (1, 4)).T
  return _tc_mlp(xu, xb, offu, offb, w1u4t, w1b4t, b1.reshape(1, H1),
                 W2.T, b2.reshape(1, H2))


# speedup vs baseline: 2.7929x; 1.0200x over previous
import functools

import jax
import jax.numpy as jnp
from jax import lax
from jax.experimental import pallas as pl
from jax.experimental.pallas import tpu as pltpu
from jax.experimental.pallas import tpu_sc as plsc

M = 1000000
B = 16384
D = 32
H1 = 64
H2 = 32

RB = 8192
G = 31
S = RB * G

NC = 2
NS = 16
NW = NC * NS
BPW = B // NW
CH = 128
NCH = BPW // CH

BLK = 2048


def _round_bf16(u):
  return (u + jnp.uint32(0x7FFF) + ((u >> jnp.uint32(16)) & jnp.uint32(1))
          ) >> jnp.uint32(16)


def _relayout_body(u0, u1, u2, u3, b0, b1, b2, b3, out_ref):
  cat_u = jnp.concatenate(
      [u0[...], u1[...], u2[...], u3[...]], axis=0)
  cat_b = jnp.concatenate(
      [b0[...], b1[...], b2[...], b3[...]], axis=0)
  bu = _round_bf16(lax.bitcast_convert_type(cat_u.T, jnp.uint32))
  bb = _round_bf16(lax.bitcast_convert_type(cat_b.T, jnp.uint32))
  packed = bu | (bb << jnp.uint32(16))
  out_ref[...] = lax.bitcast_convert_type(packed, jnp.float32)


def _tc_relayout(ttu, ttb):
  specs = [pl.BlockSpec(
      (32, RB),
      lambda g, k=k: (0, jnp.minimum((k * S) // RB + g, M // RB)))
      for k in range(4)]
  return pl.pallas_call(
      _relayout_body,
      grid=(G,),
      in_specs=specs + specs,
      out_specs=pl.BlockSpec((RB, 128), lambda g: (g, 0)),
      out_shape=jax.ShapeDtypeStruct((S, 128), jnp.float32),
  )(ttu, ttu, ttu, ttu, ttb, ttb, ttb, ttb)


def _sc_gather(p2d, lin):
  mesh = plsc.VectorSubcoreMesh(core_axis_name="c", subcore_axis_name="s")

  @functools.partial(
      pl.kernel,
      out_type=jax.ShapeDtypeStruct((B, 128), jnp.float32),
      mesh=mesh,
      compiler_params=pltpu.CompilerParams(use_tc_tiling_on_sc=False),
      scratch_types=[
          pltpu.VMEM((NCH, CH), jnp.int32),
          pltpu.VMEM((BPW, 128), jnp.float32),
          pltpu.SemaphoreType.DMA,
      ],
  )
  def gather_kernel(p_hbm, lin_hbm, x_hbm, idx_v, rows_v, sem):
    wid = lax.axis_index("s") * NC + lax.axis_index("c")
    base = wid * BPW
    pltpu.sync_copy(p_hbm.at[pl.ds(wid * NCH, NCH)], idx_v)
    cps = [
        pltpu.async_copy(lin_hbm.at[idx_v.at[j]],
                         rows_v.at[pl.ds(j * CH, CH)], sem)
        for j in range(NCH)
    ]
    for c in cps:
      c.wait()
    pltpu.sync_copy(rows_v, x_hbm.at[pl.ds(base, BPW)])

  return gather_kernel(p2d, lin)


def _mlp_body(xu_ref, xb_ref, offu_ref, offb_ref, w1u_ref, w1b_ref, b1_ref,
              w2t_ref, b2_ref, o_ref):
  lane = lax.broadcasted_iota(jnp.int32, (BLK, 128), 1)
  offu = offu_ref[...]
  offb = offb_ref[...]
  uu = lax.bitcast_convert_type(xu_ref[...], jnp.uint32)
  ub = lax.bitcast_convert_type(xb_ref[...], jnp.uint32)
  xu_vals = lax.bitcast_convert_type(uu << jnp.uint32(16), jnp.float32)
  xb_vals = lax.bitcast_convert_type(ub & jnp.uint32(0xFFFF0000), jnp.float32)
  xu = jnp.where((lane >= offu) & (lane < offu + D), xu_vals, 0.0)
  xb = jnp.where((lane >= offb) & (lane < offb + D), xb_vals, 0.0)
  h = lax.dot_general(xu, w1u_ref[...], (((1,), (0,)), ((), ())),
                      preferred_element_type=jnp.float32)
  h = h + lax.dot_general(xb, w1b_ref[...], (((1,), (0,)), ((), ())),
                          preferred_element_type=jnp.float32)
  h = jnp.maximum(h + b1_ref[...], 0.0)
  o = lax.dot_general(h, w2t_ref[...], (((1,), (0,)), ((), ())),
                      preferred_element_type=jnp.float32)
  o_ref[...] = jnp.maximum(o + b2_ref[...], 0.0)


def _tc_mlp(xu, xb, offu, offb, w1u4t, w1b4t, b1r, w2t, b2r):
  grid = (B // BLK,)
  return pl.pallas_call(
      _mlp_body,
      grid=grid,
      in_specs=[
          pl.BlockSpec((BLK, 128), lambda i: (i, 0)),
          pl.BlockSpec((BLK, 128), lambda i: (i, 0)),
          pl.BlockSpec((BLK, 1), lambda i: (i, 0)),
          pl.BlockSpec((BLK, 1), lambda i: (i, 0)),
          pl.BlockSpec((128, H1), lambda i: (0, 0)),
          pl.BlockSpec((128, H1), lambda i: (0, 0)),
          pl.BlockSpec((1, H1), lambda i: (0, 0)),
          pl.BlockSpec((H1, H2), lambda i: (0, 0)),
          pl.BlockSpec((1, H2), lambda i: (0, 0)),
      ],
      out_specs=pl.BlockSpec((BLK, H2), lambda i: (i, 0)),
      out_shape=jax.ShapeDtypeStruct((B, H2), jnp.float32),
  )(xu, xb, offu, offb, w1u4t, w1b4t, b1r, w2t, b2r)


def _split(idx):
  idx = idx.astype(jnp.int32)
  k = ((idx >= S).astype(jnp.int32) + (idx >= 2 * S).astype(jnp.int32)
       + (idx >= 3 * S).astype(jnp.int32))
  p = idx - k * S
  return p.reshape(NW * NCH, CH), (k * D).reshape(B, 1)


def kernel(user_id, book_id, user_table, book_table, W1, b1, W2, b2):
  pu2d, offu = _split(user_id)
  pb2d, offb = _split(book_id)
  lin = _tc_relayout(user_table.T, book_table.T)
  xu = _sc_gather(pu2d, lin)
  xb = _sc_gather(pb2d, lin)
  w1u4t = jnp.tile(W1[:, :D], (1, 4)).T
  w1b4t = jnp.tile(W1[:, D:], (1, 4)).T
  return _tc_mlp(xu, xb, offu, offb, w1u4t, w1b4t, b1.reshape(1, H1),
                 W2.T, b2.reshape(1, H2))


# speedup vs baseline: 2.9167x; 1.0443x over previous
import functools

import jax
import jax.numpy as jnp
from jax import lax
from jax.experimental import pallas as pl
from jax.experimental.pallas import tpu as pltpu
from jax.experimental.pallas import tpu_sc as plsc

M = 1000000
B = 16384
D = 32
H1 = 64
H2 = 32

RB = 8192
G = 31
S = RB * G

NC = 2
NS = 16
NW = NC * NS
BPW = B // NW
CH = 128
NCH = BPW // CH

BLK = 2048


def _round_bf16(u):
  return (u + jnp.uint32(0x7FFF) + ((u >> jnp.uint32(16)) & jnp.uint32(1))
          ) >> jnp.uint32(16)


def _relayout_body(u0, u1, u2, u3, b0, b1, b2, b3, out_ref):
  cat_u = jnp.concatenate(
      [u0[...], u1[...], u2[...], u3[...]], axis=0)
  cat_b = jnp.concatenate(
      [b0[...], b1[...], b2[...], b3[...]], axis=0)
  bu = _round_bf16(lax.bitcast_convert_type(cat_u.T, jnp.uint32))
  bb = _round_bf16(lax.bitcast_convert_type(cat_b.T, jnp.uint32))
  packed = bu | (bb << jnp.uint32(16))
  out_ref[...] = lax.bitcast_convert_type(packed, jnp.float32)


def _tc_relayout(ttu, ttb):
  specs = [pl.BlockSpec(
      (32, RB),
      lambda g, k=k: (0, jnp.minimum((k * S) // RB + g, M // RB)))
      for k in range(4)]
  return pl.pallas_call(
      _relayout_body,
      grid=(G,),
      in_specs=specs + specs,
      out_specs=pl.BlockSpec((RB, 128), lambda g: (g, 0)),
      out_shape=jax.ShapeDtypeStruct((S, 128), jnp.float32),
  )(ttu, ttu, ttu, ttu, ttb, ttb, ttb, ttb)


def _sc_gather(pu2d, pb2d, lin):
  mesh = plsc.VectorSubcoreMesh(core_axis_name="c", subcore_axis_name="s")

  @functools.partial(
      pl.kernel,
      out_type=(
          jax.ShapeDtypeStruct((B, 128), jnp.float32),
          jax.ShapeDtypeStruct((B, 128), jnp.float32),
      ),
      mesh=mesh,
      compiler_params=pltpu.CompilerParams(use_tc_tiling_on_sc=False),
      scratch_types=[
          pltpu.VMEM((NCH, CH), jnp.int32),
          pltpu.VMEM((NCH, CH), jnp.int32),
          pltpu.VMEM((BPW, 128), jnp.float32),
          pltpu.SemaphoreType.DMA,
      ],
  )
  def gather_kernel(pu_hbm, pb_hbm, lin_hbm, xu_hbm, xb_hbm,
                    uidx_v, bidx_v, rows_v, sem):
    wid = lax.axis_index("s") * NC + lax.axis_index("c")
    base = wid * BPW
    pltpu.sync_copy(pu_hbm.at[pl.ds(wid * NCH, NCH)], uidx_v)
    pltpu.sync_copy(pb_hbm.at[pl.ds(wid * NCH, NCH)], bidx_v)
    ucps = [
        pltpu.async_copy(lin_hbm.at[uidx_v.at[j]],
                         rows_v.at[pl.ds(j * CH, CH)], sem)
        for j in range(NCH)
    ]
    for c in ucps:
      c.wait()
    pltpu.sync_copy(rows_v, xu_hbm.at[pl.ds(base, BPW)])
    bcps = [
        pltpu.async_copy(lin_hbm.at[bidx_v.at[j]],
                         rows_v.at[pl.ds(j * CH, CH)], sem)
        for j in range(NCH)
    ]
    for c in bcps:
      c.wait()
    pltpu.sync_copy(rows_v, xb_hbm.at[pl.ds(base, BPW)])

  return gather_kernel(pu2d, pb2d, lin)


def _mlp_body(xu_ref, xb_ref, mu_ref, mb_ref, w1u_ref, w1b_ref, b1_ref,
              w2t_ref, b2_ref, o_ref):
  uu = lax.bitcast_convert_type(xu_ref[...], jnp.uint32)
  ub = lax.bitcast_convert_type(xb_ref[...], jnp.uint32)
  xu_vals = lax.bitcast_convert_type(uu << jnp.uint32(16), jnp.float32)
  xb_vals = lax.bitcast_convert_type(ub & jnp.uint32(0xFFFF0000), jnp.float32)
  xu = jnp.where(mu_ref[...] != 0, xu_vals, 0.0)
  xb = jnp.where(mb_ref[...] != 0, xb_vals, 0.0)
  h = lax.dot_general(xu, w1u_ref[...], (((1,), (0,)), ((), ())),
                      preferred_element_type=jnp.float32)
  h = h + lax.dot_general(xb, w1b_ref[...], (((1,), (0,)), ((), ())),
                          preferred_element_type=jnp.float32)
  h = jnp.maximum(h + b1_ref[...], 0.0)
  o = lax.dot_general(h, w2t_ref[...], (((1,), (0,)), ((), ())),
                      preferred_element_type=jnp.float32)
  o_ref[...] = jnp.maximum(o + b2_ref[...], 0.0)


def _tc_mlp(xu, xb, mu, mb, w1u4t, w1b4t, b1r, w2t, b2r):
  grid = (B // BLK,)
  return pl.pallas_call(
      _mlp_body,
      grid=grid,
      in_specs=[
          pl.BlockSpec((BLK, 128), lambda i: (i, 0)),
          pl.BlockSpec((BLK, 128), lambda i: (i, 0)),
          pl.BlockSpec((BLK, 128), lambda i: (i, 0)),
          pl.BlockSpec((BLK, 128), lambda i: (i, 0)),
          pl.BlockSpec((128, H1), lambda i: (0, 0)),
          pl.BlockSpec((128, H1), lambda i: (0, 0)),
          pl.BlockSpec((1, H1), lambda i: (0, 0)),
          pl.BlockSpec((H1, H2), lambda i: (0, 0)),
          pl.BlockSpec((1, H2), lambda i: (0, 0)),
      ],
      out_specs=pl.BlockSpec((BLK, H2), lambda i: (i, 0)),
      out_shape=jax.ShapeDtypeStruct((B, H2), jnp.float32),
  )(xu, xb, mu, mb, w1u4t, w1b4t, b1r, w2t, b2r)


def _split(idx):
  idx = idx.astype(jnp.int32)
  k = ((idx >= S).astype(jnp.int32) + (idx >= 2 * S).astype(jnp.int32)
       + (idx >= 3 * S).astype(jnp.int32))
  p = idx - k * S
  off = (k * D).reshape(B, 1)
  lanes = jnp.arange(128, dtype=jnp.int32)[None, :]
  mask = ((lanes >= off) & (lanes < off + D)).astype(jnp.int8)
  return p.reshape(NW * NCH, CH), mask


def kernel(user_id, book_id, user_table, book_table, W1, b1, W2, b2):
  pu2d, mu = _split(user_id)
  pb2d, mb = _split(book_id)
  lin = _tc_relayout(user_table.T, book_table.T)
  xu, xb = _sc_gather(pu2d, pb2d, lin)
  w1u4t = jnp.tile(W1[:, :D], (1, 4)).T
  w1b4t = jnp.tile(W1[:, D:], (1, 4)).T
  return _tc_mlp(xu, xb, mu, mb, w1u4t, w1b4t, b1.reshape(1, H1),
                 W2.T, b2.reshape(1, H2))


# speedup vs baseline: 2.9555x; 1.0133x over previous
import functools

import jax
import jax.numpy as jnp
from jax import lax
from jax.experimental import pallas as pl
from jax.experimental.pallas import tpu as pltpu
from jax.experimental.pallas import tpu_sc as plsc

M = 1000000
B = 16384
D = 32
H1 = 64
H2 = 32

RB = 8192
G = 31
S = RB * G

NC = 2
NS = 16
NW = NC * NS
BPW = B // NW
CH = 128
NCH = BPW // CH

BLK = 4096


def _round_bf16(u):
  return (u + jnp.uint32(0x7FFF) + ((u >> jnp.uint32(16)) & jnp.uint32(1))
          ) >> jnp.uint32(16)


def _relayout_body(u0, u1, u2, u3, b0, b1, b2, b3, out_ref):
  cat_u = jnp.concatenate(
      [u0[...], u1[...], u2[...], u3[...]], axis=0)
  cat_b = jnp.concatenate(
      [b0[...], b1[...], b2[...], b3[...]], axis=0)
  bu = _round_bf16(lax.bitcast_convert_type(cat_u.T, jnp.uint32))
  bb = _round_bf16(lax.bitcast_convert_type(cat_b.T, jnp.uint32))
  packed = bu | (bb << jnp.uint32(16))
  out_ref[...] = lax.bitcast_convert_type(packed, jnp.float32)


def _tc_relayout(ttu, ttb):
  specs = [pl.BlockSpec(
      (32, RB),
      lambda g, k=k: (0, jnp.minimum((k * S) // RB + g, M // RB)))
      for k in range(4)]
  return pl.pallas_call(
      _relayout_body,
      grid=(G,),
      in_specs=specs + specs,
      out_specs=pl.BlockSpec((RB, 128), lambda g: (g, 0)),
      out_shape=jax.ShapeDtypeStruct((S, 128), jnp.float32),
  )(ttu, ttu, ttu, ttu, ttb, ttb, ttb, ttb)


def _sc_gather(pu2d, pb2d, lin):
  mesh = plsc.VectorSubcoreMesh(core_axis_name="c", subcore_axis_name="s")

  @functools.partial(
      pl.kernel,
      out_type=(
          jax.ShapeDtypeStruct((B, 128), jnp.float32),
          jax.ShapeDtypeStruct((B, 128), jnp.float32),
      ),
      mesh=mesh,
      compiler_params=pltpu.CompilerParams(use_tc_tiling_on_sc=False),
      scratch_types=[
          pltpu.VMEM((NCH, CH), jnp.int32),
          pltpu.VMEM((NCH, CH), jnp.int32),
          pltpu.VMEM((BPW, 128), jnp.float32),
          pltpu.SemaphoreType.DMA,
      ],
  )
  def gather_kernel(pu_hbm, pb_hbm, lin_hbm, xu_hbm, xb_hbm,
                    uidx_v, bidx_v, rows_v, sem):
    wid = lax.axis_index("s") * NC + lax.axis_index("c")
    base = wid * BPW
    pltpu.sync_copy(pu_hbm.at[pl.ds(wid * NCH, NCH)], uidx_v)
    pltpu.sync_copy(pb_hbm.at[pl.ds(wid * NCH, NCH)], bidx_v)
    ucps = [
        pltpu.async_copy(lin_hbm.at[uidx_v.at[j]],
                         rows_v.at[pl.ds(j * CH, CH)], sem)
        for j in range(NCH)
    ]
    for c in ucps:
      c.wait()
    pltpu.sync_copy(rows_v, xu_hbm.at[pl.ds(base, BPW)])
    bcps = [
        pltpu.async_copy(lin_hbm.at[bidx_v.at[j]],
                         rows_v.at[pl.ds(j * CH, CH)], sem)
        for j in range(NCH)
    ]
    for c in bcps:
      c.wait()
    pltpu.sync_copy(rows_v, xb_hbm.at[pl.ds(base, BPW)])

  return gather_kernel(pu2d, pb2d, lin)


def _mlp_body(xu_ref, xb_ref, mu_ref, mb_ref, w1u_ref, w1b_ref, b1_ref,
              w2t_ref, b2_ref, o_ref):
  uu = lax.bitcast_convert_type(xu_ref[...], jnp.uint32)
  ub = lax.bitcast_convert_type(xb_ref[...], jnp.uint32)
  xu_vals = lax.bitcast_convert_type(uu << jnp.uint32(16), jnp.float32)
  xb_vals = lax.bitcast_convert_type(ub & jnp.uint32(0xFFFF0000), jnp.float32)
  xu = jnp.where(mu_ref[...] != 0, xu_vals, 0.0)
  xb = jnp.where(mb_ref[...] != 0, xb_vals, 0.0)
  h = lax.dot_general(xu, w1u_ref[...], (((1,), (0,)), ((), ())),
                      preferred_element_type=jnp.float32)
  h = h + lax.dot_general(xb, w1b_ref[...], (((1,), (0,)), ((), ())),
                          preferred_element_type=jnp.float32)
  h = jnp.maximum(h + b1_ref[...], 0.0)
  o = lax.dot_general(h, w2t_ref[...], (((1,), (0,)), ((), ())),
                      preferred_element_type=jnp.float32)
  o_ref[...] = jnp.maximum(o + b2_ref[...], 0.0)


def _tc_mlp(xu, xb, mu, mb, w1u4t, w1b4t, b1r, w2t, b2r):
  grid = (B // BLK,)
  return pl.pallas_call(
      _mlp_body,
      grid=grid,
      in_specs=[
          pl.BlockSpec((BLK, 128), lambda i: (i, 0)),
          pl.BlockSpec((BLK, 128), lambda i: (i, 0)),
          pl.BlockSpec((BLK, 128), lambda i: (i, 0)),
          pl.BlockSpec((BLK, 128), lambda i: (i, 0)),
          pl.BlockSpec((128, H1), lambda i: (0, 0)),
          pl.BlockSpec((128, H1), lambda i: (0, 0)),
          pl.BlockSpec((1, H1), lambda i: (0, 0)),
          pl.BlockSpec((H1, H2), lambda i: (0, 0)),
          pl.BlockSpec((1, H2), lambda i: (0, 0)),
      ],
      out_specs=pl.BlockSpec((BLK, H2), lambda i: (i, 0)),
      out_shape=jax.ShapeDtypeStruct((B, H2), jnp.float32),
  )(xu, xb, mu, mb, w1u4t, w1b4t, b1r, w2t, b2r)


def _split(idx):
  idx = idx.astype(jnp.int32)
  k = ((idx >= S).astype(jnp.int32) + (idx >= 2 * S).astype(jnp.int32)
       + (idx >= 3 * S).astype(jnp.int32))
  p = idx - k * S
  off = (k * D).reshape(B, 1)
  lanes = jnp.arange(128, dtype=jnp.int32)[None, :]
  mask = ((lanes >= off) & (lanes < off + D)).astype(jnp.int8)
  return p.reshape(NW * NCH, CH), mask


def kernel(user_id, book_id, user_table, book_table, W1, b1, W2, b2):
  pu2d, mu = _split(user_id)
  pb2d, mb = _split(book_id)
  lin = _tc_relayout(user_table.T, book_table.T)
  xu, xb = _sc_gather(pu2d, pb2d, lin)
  w1u4t = jnp.tile(W1[:, :D], (1, 4)).T
  w1b4t = jnp.tile(W1[:, D:], (1, 4)).T
  return _tc_mlp(xu, xb, mu, mb, w1u4t, w1b4t, b1.reshape(1, H1),
                 W2.T, b2.reshape(1, H2))
